# baseline (speedup 1.0000x reference)
import functools

import jax
import jax.numpy as jnp
from jax import lax
from jax.experimental import pallas as pl
from jax.experimental.pallas import tpu as pltpu
from jax.experimental.pallas import tpu_sc as plsc

_BM = 1024


_NW = 32
_LANES = 16


def _sc_gather_body(tablet_ref, ids_ref, out_ref, table_v, ids_v, out_v):
    nc = 2
    wid = lax.axis_index("s") * nc + lax.axis_index("c")
    batch = ids_v.shape[0]
    n_emb = 8
    seq_len = tablet_ref.shape[0] // _LANES
    s_per_w = seq_len // _NW

    pltpu.sync_copy(
        tablet_ref.at[pl.ds(wid * s_per_w * _LANES, s_per_w * _LANES)], table_v
    )
    pltpu.sync_copy(ids_ref, ids_v)

    e_vecs = [ids_v[pl.ds(v * _LANES, _LANES)] for v in range(batch // _LANES)]

    def body(s_local, carry):
        tv = table_v[pl.ds(s_local * _LANES, _LANES)]
        tvb = [jnp.broadcast_to(tv[e], (_LANES,)) for e in range(n_emb)]
        for v in range(batch // _LANES):
            val = tvb[0]
            for e in range(1, n_emb):
                val = jnp.where(e_vecs[v] == e, tvb[e], val)
            out_v[pl.ds(s_local * batch + v * _LANES, _LANES)] = val
        return carry

    lax.fori_loop(0, s_per_w, body, jnp.int32(0))
    pltpu.sync_copy(out_v, out_ref.at[pl.ds(wid * s_per_w * batch, s_per_w * batch)])


def _sc_gather(degree_counts_by_id, embodiment_ids):
    batch = embodiment_ids.shape[0]
    n_emb, seq_len = degree_counts_by_id.shape
    tablet = jnp.pad(
        degree_counts_by_id.T, ((0, 0), (0, _LANES - n_emb))
    ).reshape(-1)
    s_per_w = seq_len // _NW
    mesh = plsc.VectorSubcoreMesh(core_axis_name="c", subcore_axis_name="s")
    return pl.kernel(
        _sc_gather_body,
        out_type=jax.ShapeDtypeStruct((seq_len * batch,), jnp.int32),
        mesh=mesh,
        scratch_types=[
            pltpu.VMEM((s_per_w * _LANES,), jnp.int32),
            pltpu.VMEM((batch,), jnp.int32),
            pltpu.VMEM((s_per_w * batch,), jnp.int32),
        ],
    )(tablet, embodiment_ids)


def _tc_body(dc_ref, emb_ref, tok_ref, out_ref):
    bm = tok_ref.shape[0]
    bs = dc_ref.shape[1]
    nb = dc_ref.shape[2]
    n_rows = emb_ref.shape[0]

    dc_sb = dc_ref[0].astype(jnp.float32)

    rs_io = lax.broadcasted_iota(jnp.int32, (bm, bs), 0)
    s_io = lax.broadcasted_iota(jnp.int32, (bm, bs), 1)
    rep_s = (rs_io // nb == s_io).astype(jnp.float32)
    tmp = jnp.dot(rep_s, dc_sb, preferred_element_type=jnp.float32)

    r_io = lax.broadcasted_iota(jnp.int32, (bm, nb), 0)
    b_io = lax.broadcasted_iota(jnp.int32, (bm, nb), 1)
    mask_b = (r_io % nb == b_io).astype(jnp.float32)
    idx = jnp.sum(tmp * mask_b, axis=1, keepdims=True)

    k_io = lax.broadcasted_iota(jnp.int32, (bm, n_rows), 1).astype(jnp.float32)
    oh = (idx == k_io).astype(jnp.bfloat16)
    pe = jnp.dot(oh, emb_ref[...], preferred_element_type=jnp.float32)
    out_ref[...] = tok_ref[...] + pe


def kernel(tokens, embodiment_ids, degree_counts_by_id, embedding):
    seq_len, batch, d_model = tokens.shape
    n_rows = embedding.shape[0]
    m = seq_len * batch
    bs = _BM // batch

    degree_counts = _sc_gather(degree_counts_by_id, embodiment_ids)
    dc3 = degree_counts.reshape(m // _BM, bs, batch)

    tok2 = tokens.reshape(m, d_model)
    emb_bf = embedding.astype(jnp.bfloat16)

    out2 = pl.pallas_call(
        _tc_body,
        grid=(m // _BM,),
        in_specs=[
            pl.BlockSpec((1, bs, batch), lambda i: (i, 0, 0)),
            pl.BlockSpec((n_rows, d_model), lambda i: (0, 0)),
            pl.BlockSpec((_BM, d_model), lambda i: (i, 0)),
        ],
        out_specs=pl.BlockSpec((_BM, d_model), lambda i: (i, 0)),
        out_shape=jax.ShapeDtypeStruct((m, d_model), jnp.float32),
    )(dc3, emb_bf, tok2)
    return out2.reshape(seq_len, batch, d_model)


# speedup vs baseline: 1.0130x; 1.0130x over previous
import jax
import jax.numpy as jnp
from jax import lax
from jax.experimental import pallas as pl
from jax.experimental.pallas import tpu as pltpu
from jax.experimental.pallas import tpu_sc as plsc

_BM = 2048


_NW = 32
_LANES = 16


def _sc_gather_body(tablet_ref, ids_ref, out_ref, table_v, ids_v, out_v):
    nc = 2
    wid = lax.axis_index("s") * nc + lax.axis_index("c")
    batch = ids_v.shape[0]
    n_emb = 8
    seq_len = tablet_ref.shape[0] // _LANES
    s_per_w = seq_len // _NW

    pltpu.sync_copy(
        tablet_ref.at[pl.ds(wid * s_per_w * _LANES, s_per_w * _LANES)], table_v
    )
    pltpu.sync_copy(ids_ref, ids_v)

    e_vecs = [ids_v[pl.ds(v * _LANES, _LANES)] for v in range(batch // _LANES)]

    def body(s_local, carry):
        tv = table_v[pl.ds(s_local * _LANES, _LANES)]
        tvb = [jnp.broadcast_to(tv[e], (_LANES,)) for e in range(n_emb)]
        for v in range(batch // _LANES):
            val = tvb[0]
            for e in range(1, n_emb):
                val = jnp.where(e_vecs[v] == e, tvb[e], val)
            out_v[pl.ds(s_local * batch + v * _LANES, _LANES)] = val
        return carry

    lax.fori_loop(0, s_per_w, body, jnp.int32(0))
    pltpu.sync_copy(out_v, out_ref.at[pl.ds(wid * s_per_w * batch, s_per_w * batch)])


def _sc_gather(degree_counts_by_id, embodiment_ids):
    batch = embodiment_ids.shape[0]
    n_emb, seq_len = degree_counts_by_id.shape
    tablet = jnp.pad(
        degree_counts_by_id.T, ((0, 0), (0, _LANES - n_emb))
    ).reshape(-1)
    s_per_w = seq_len // _NW
    mesh = plsc.VectorSubcoreMesh(core_axis_name="c", subcore_axis_name="s")
    return pl.kernel(
        _sc_gather_body,
        out_type=jax.ShapeDtypeStruct((seq_len * batch,), jnp.int32),
        mesh=mesh,
        scratch_types=[
            pltpu.VMEM((s_per_w * _LANES,), jnp.int32),
            pltpu.VMEM((batch,), jnp.int32),
            pltpu.VMEM((s_per_w * batch,), jnp.int32),
        ],
    )(tablet, embodiment_ids)


def _tc_body(dc_ref, emb_ref, tok_ref, out_ref):
    bm = tok_ref.shape[0]
    bs = dc_ref.shape[1]
    nb = dc_ref.shape[2]
    n_rows = emb_ref.shape[0]

    dc_sb = dc_ref[0].astype(jnp.float32)

    rs_io = lax.broadcasted_iota(jnp.int32, (bm, bs), 0)
    s_io = lax.broadcasted_iota(jnp.int32, (bm, bs), 1)
    rep_s = (rs_io // nb == s_io).astype(jnp.float32)
    tmp = jnp.dot(rep_s, dc_sb, preferred_element_type=jnp.float32)

    r_io = lax.broadcasted_iota(jnp.int32, (bm, nb), 0)
    b_io = lax.broadcasted_iota(jnp.int32, (bm, nb), 1)
    mask_b = (r_io % nb == b_io).astype(jnp.float32)
    idx = jnp.sum(tmp * mask_b, axis=1, keepdims=True)

    k_io = lax.broadcasted_iota(jnp.int32, (bm, n_rows), 1).astype(jnp.float32)
    oh = (idx == k_io).astype(jnp.bfloat16)
    pe = jnp.dot(oh, emb_ref[...], preferred_element_type=jnp.float32)
    out_ref[...] = tok_ref[...] + pe


def kernel(tokens, embodiment_ids, degree_counts_by_id, embedding):
    seq_len, batch, d_model = tokens.shape
    n_rows = embedding.shape[0]
    m = seq_len * batch
    bs = _BM // batch

    degree_counts = _sc_gather(degree_counts_by_id, embodiment_ids)
    dc3 = degree_counts.reshape(m // _BM, bs, batch)

    tok2 = tokens.reshape(m, d_model)
    emb_bf = embedding.astype(jnp.bfloat16)

    out2 = pl.pallas_call(
        _tc_body,
        grid=(m // _BM,),
        in_specs=[
            pl.BlockSpec((1, bs, batch), lambda i: (i, 0, 0)),
            pl.BlockSpec((n_rows, d_model), lambda i: (0, 0)),
            pl.BlockSpec((_BM, d_model), lambda i: (i, 0)),
        ],
        out_specs=pl.BlockSpec((_BM, d_model), lambda i: (i, 0)),
        out_shape=jax.ShapeDtypeStruct((m, d_model), jnp.float32),
    )(dc3, emb_bf, tok2)
    return out2.reshape(seq_len, batch, d_model)
